# baseline (speedup 1.0000x reference)
import functools

import numpy as np
import jax
import jax.numpy as jnp
from jax import lax
from jax.experimental import pallas as pl
from jax.experimental.pallas import tpu as pltpu
from jax.experimental.pallas import tpu_sc as plsc

_SEGMENTS = ((0, 128, 1), (128, 128, 3), (512, 128, 5), (1152, 128, 7))
_D = 2048
_Z = 16384
_NW = 32
_ROWS_PER_W = _Z // _NW
_B = 16
_RG = 4


def _build_perm() -> np.ndarray:
    p = np.empty(_D, np.int32)
    for off, u, v in _SEGMENTS:
        for k in range(v):
            for i in range(u):
                p[off + k * u + i] = off + i * v + k
    return p


_PERM = _build_perm()


@jax.jit
def _sc_transpose(x_flat, perm):
    mesh = plsc.VectorSubcoreMesh(core_axis_name="c", subcore_axis_name="s")

    @functools.partial(
        pl.kernel,
        mesh=mesh,
        out_type=jax.ShapeDtypeStruct((_Z * _D,), jnp.float32),
        scratch_types=[
            pltpu.VMEM((_D,), jnp.int32),
            pltpu.VMEM((_B * _D,), jnp.float32),
            pltpu.VMEM((_B * _D,), jnp.float32),
        ],
        compiler_params=pltpu.CompilerParams(needs_layout_passes=False),
    )
    def k(x_hbm, perm_hbm, out_hbm, perm_v, in_v, out_v):
        cid = lax.axis_index("c")
        sid = lax.axis_index("s")
        wid = sid * 2 + cid
        base = wid * (_ROWS_PER_W * _D)
        pltpu.sync_copy(perm_hbm, perm_v)

        def blk(g, carry):
            off = base + g * (_B * _D)
            pltpu.sync_copy(x_hbm.at[pl.ds(off, _B * _D)], in_v)

            def grp(t, carry2):
                r0 = t * (_RG * _D)
                for j in range(_D // 16):
                    pj = perm_v[pl.ds(j * 16, 16)]
                    for rr in range(_RG):
                        row = r0 + rr * _D
                        vals = plsc.load_gather(in_v, [pj + row])
                        out_v[pl.ds(row + j * 16, 16)] = vals
                return carry2

            lax.fori_loop(0, _B // _RG, grp, 0)
            pltpu.sync_copy(out_v, out_hbm.at[pl.ds(off, _B * _D)])
            return carry

        lax.fori_loop(0, _ROWS_PER_W // _B, blk, 0)

    return k(x_flat, perm)


def kernel(x):
    x_flat = jnp.reshape(x, (-1,))
    perm = jnp.asarray(_PERM)
    out = _sc_transpose(x_flat, perm)
    return jnp.reshape(out, (_Z, _D))


# speedup vs baseline: 1.0329x; 1.0329x over previous
import functools

import numpy as np
import jax
import jax.numpy as jnp
from jax import lax
from jax.experimental import pallas as pl
from jax.experimental.pallas import tpu as pltpu
from jax.experimental.pallas import tpu_sc as plsc

_SEGMENTS = ((0, 128, 1), (128, 128, 3), (512, 128, 5), (1152, 128, 7))
_D = 2048
_Z = 16384
_NW = 32
_ROWS_PER_W = _Z // _NW
_B = 8
_NBLK = _ROWS_PER_W // _B
_RG = 4


def _build_perm() -> np.ndarray:
    p = np.empty(_D, np.int32)
    for off, u, v in _SEGMENTS:
        for k in range(v):
            for i in range(u):
                p[off + k * u + i] = off + i * v + k
    return p


_PERM = _build_perm()


@jax.jit
def _sc_transpose(x_flat, perm):
    mesh = plsc.VectorSubcoreMesh(core_axis_name="c", subcore_axis_name="s")

    @functools.partial(
        pl.kernel,
        mesh=mesh,
        out_type=jax.ShapeDtypeStruct((_Z * _D,), jnp.float32),
        scratch_types=[
            pltpu.VMEM((_D,), jnp.int32),
            pltpu.VMEM((_B * _D,), jnp.float32),
            pltpu.VMEM((_B * _D,), jnp.float32),
            pltpu.VMEM((_B * _D,), jnp.float32),
            pltpu.VMEM((_B * _D,), jnp.float32),
            pltpu.SemaphoreType.DMA,
            pltpu.SemaphoreType.DMA,
            pltpu.SemaphoreType.DMA,
            pltpu.SemaphoreType.DMA,
        ],
        compiler_params=pltpu.CompilerParams(needs_layout_passes=False),
    )
    def k(x_hbm, perm_hbm, out_hbm, perm_v, in_v0, in_v1, out_v0, out_v1,
          sem_in0, sem_in1, sem_out0, sem_out1):
        cid = lax.axis_index("c")
        sid = lax.axis_index("s")
        wid = sid * 2 + cid
        base = wid * (_ROWS_PER_W * _D)
        sem_in = (sem_in0, sem_in1)
        sem_out = (sem_out0, sem_out1)
        in_v = (in_v0, in_v1)
        out_v = (out_v0, out_v1)

        pltpu.sync_copy(perm_hbm, perm_v)

        def in_copy(g, b):
            return pltpu.make_async_copy(
                x_hbm.at[pl.ds(base + g * (_B * _D), _B * _D)],
                in_v[b], sem_in[b])

        def out_copy(g, b):
            return pltpu.make_async_copy(
                out_v[b],
                out_hbm.at[pl.ds(base + g * (_B * _D), _B * _D)], sem_out[b])

        def compute(b):
            def grp(t, carry):
                r0 = t * (_RG * _D)
                for j in range(_D // 16):
                    pj = perm_v[pl.ds(j * 16, 16)]
                    for rr in range(_RG):
                        row = r0 + rr * _D
                        vals = plsc.load_gather(in_v[b], [pj + row])
                        out_v[b][pl.ds(row + j * 16, 16)] = vals
                return carry
            lax.fori_loop(0, _B // _RG, grp, 0)

        in_copy(0, 0).start()
        in_copy(1, 1).start()

        def body(h, carry):
            for b in range(2):
                g = h * 2 + b
                in_copy(g, b).wait()

                @pl.when(g >= 2)
                def _():
                    out_copy(g - 2, b).wait()

                compute(b)
                out_copy(g, b).start()

                @pl.when(g + 2 < _NBLK)
                def _():
                    in_copy(g + 2, b).start()
            return carry

        lax.fori_loop(0, _NBLK // 2, body, 0)
        out_copy(_NBLK - 2, 0).wait()
        out_copy(_NBLK - 1, 1).wait()

    return k(x_flat, perm)


def kernel(x):
    x_flat = jnp.reshape(x, (-1,))
    perm = jnp.asarray(_PERM)
    out = _sc_transpose(x_flat, perm)
    return jnp.reshape(out, (_Z, _D))


# speedup vs baseline: 2.1470x; 2.0785x over previous
import functools

import numpy as np
import jax
import jax.numpy as jnp
from jax import lax
from jax.experimental import pallas as pl
from jax.experimental.pallas import tpu as pltpu
from jax.experimental.pallas import tpu_sc as plsc

_SEGMENTS = ((0, 128, 1), (128, 128, 3), (512, 128, 5), (1152, 128, 7))
_D = 2048
_Z = 16384
_NW = 32
_ROWS_PER_W = _Z // _NW
_B = 8
_NBLK = _ROWS_PER_W // _B
_RG = 4


def _build_perm() -> np.ndarray:
    p = np.empty(_D, np.int32)
    for off, u, v in _SEGMENTS:
        for k in range(v):
            for i in range(u):
                p[off + k * u + i] = off + i * v + k
    return p


_PERM = _build_perm()


@jax.jit
def _sc_transpose(x_flat, perm):
    mesh = plsc.VectorSubcoreMesh(core_axis_name="c", subcore_axis_name="s")

    @functools.partial(
        pl.kernel,
        mesh=mesh,
        out_type=jax.ShapeDtypeStruct((_Z * _D,), jnp.float32),
        scratch_types=[
            pltpu.VMEM((_D,), jnp.int32),
            pltpu.VMEM((_B * _D,), jnp.int32),
            pltpu.VMEM((_B * _D,), jnp.float32),
            pltpu.VMEM((_B * _D,), jnp.float32),
            pltpu.VMEM((_B * _D,), jnp.float32),
            pltpu.VMEM((_B * _D,), jnp.float32),
            pltpu.SemaphoreType.DMA,
            pltpu.SemaphoreType.DMA,
            pltpu.SemaphoreType.DMA,
            pltpu.SemaphoreType.DMA,
        ],
        compiler_params=pltpu.CompilerParams(needs_layout_passes=False),
    )
    def k(x_hbm, perm_hbm, out_hbm, perm_v, idx_tab,
          in_v0, in_v1, out_v0, out_v1,
          sem_in0, sem_in1, sem_out0, sem_out1):
        cid = lax.axis_index("c")
        sid = lax.axis_index("s")
        wid = sid * 2 + cid
        base = wid * (_ROWS_PER_W * _D)
        sem_in = (sem_in0, sem_in1)
        sem_out = (sem_out0, sem_out1)
        in_v = (in_v0, in_v1)
        out_v = (out_v0, out_v1)

        pltpu.sync_copy(perm_hbm, perm_v)

        def mk_tab(j, carry):
            pj = perm_v[pl.ds(j * 16, 16)]
            for r in range(_B):
                idx_tab[pl.ds(r * _D + j * 16, 16)] = pj + r * _D
            return carry
        lax.fori_loop(0, _D // 16, mk_tab, 0)

        def in_copy(g, b):
            return pltpu.make_async_copy(
                x_hbm.at[pl.ds(base + g * (_B * _D), _B * _D)],
                in_v[b], sem_in[b])

        def out_copy(g, b):
            return pltpu.make_async_copy(
                out_v[b],
                out_hbm.at[pl.ds(base + g * (_B * _D), _B * _D)], sem_out[b])

        def compute(b):
            @plsc.parallel_loop(0, _B * _D, 16, unroll=16)
            def _(i):
                vals = plsc.load_gather(in_v[b], [idx_tab[pl.ds(i, 16)]])
                out_v[b][pl.ds(i, 16)] = vals

        in_copy(0, 0).start()
        in_copy(1, 1).start()

        def body(h, carry):
            for b in range(2):
                g = h * 2 + b
                in_copy(g, b).wait()

                @pl.when(g >= 2)
                def _():
                    out_copy(g - 2, b).wait()

                compute(b)
                out_copy(g, b).start()

                @pl.when(g + 2 < _NBLK)
                def _():
                    in_copy(g + 2, b).start()
            return carry

        lax.fori_loop(0, _NBLK // 2, body, 0)
        out_copy(_NBLK - 2, 0).wait()
        out_copy(_NBLK - 1, 1).wait()

    return k(x_flat, perm)


def kernel(x):
    x_flat = jnp.reshape(x, (-1,))
    perm = jnp.asarray(_PERM)
    out = _sc_transpose(x_flat, perm)
    return jnp.reshape(out, (_Z, _D))
